# baseline (speedup 1.0000x reference)
import functools

import jax
import jax.numpy as jnp
from jax import lax
from jax.experimental import pallas as pl
from jax.experimental.pallas import tpu as pltpu
from jax.experimental.pallas import tpu_sc as plsc

_TABLE_SIZE = 2048


def _sc_gather_build(n_rows, row_words, n_workers, nc):
    rows_per_w = n_rows // n_workers
    mesh = plsc.VectorSubcoreMesh(core_axis_name="c", subcore_axis_name="s")

    @functools.partial(
        pl.kernel,
        out_type=jax.ShapeDtypeStruct((n_rows, row_words), jnp.int32),
        mesh=mesh,
        scratch_types=[
            pltpu.VMEM((rows_per_w,), jnp.int32),
            pltpu.VMEM((rows_per_w, row_words), jnp.int32),
            pltpu.SemaphoreType.DMA,
        ],
    )
    def sc_gather(table_hbm, idx_hbm, out_hbm, idx_v, rows_v, sem):
        wid = lax.axis_index("s") * nc + lax.axis_index("c")
        base = wid * rows_per_w
        pltpu.sync_copy(idx_hbm.at[pl.ds(base, rows_per_w)], idx_v)
        pltpu.async_copy(table_hbm.at[idx_v], rows_v, sem).wait()
        pltpu.sync_copy(rows_v, out_hbm.at[pl.ds(base, rows_per_w)])

    return sc_gather


def _bcast_body(ids_ref, inv_ref, g_ref, ocos_ref, osin_ref):
    ids = ids_ref[...]
    in_table = jnp.logical_and(jnp.max(ids) < _TABLE_SIZE, jnp.min(ids) >= 0)
    scale = jnp.where(in_table, jnp.float32(1.0),
                      jnp.float32(1.0) + jnp.sum(inv_ref[...]))
    g = g_ref[...] * scale
    d = ocos_ref.shape[-1]
    ocos_ref[...] = g[:, :, :d].reshape(ocos_ref.shape)
    osin_ref[...] = g[:, :, d:].reshape(osin_ref.shape)


def kernel(x, lookup_cos, lookup_sin, inv_freq, position_ids):
    B, H, S, D = x.shape
    T = lookup_cos.shape[0]
    pos = position_ids.astype(jnp.int32)
    n_rows = B * S
    half_words = D // 2

    cos_i32 = lax.bitcast_convert_type(
        lookup_cos.reshape(T, half_words, 2), jnp.int32)
    sin_i32 = lax.bitcast_convert_type(
        lookup_sin.reshape(T, half_words, 2), jnp.int32)
    packed = jnp.concatenate([cos_i32, sin_i32], axis=1)
    idx_flat = jnp.clip(pos.reshape(n_rows), 0, T - 1)

    info = plsc.get_sparse_core_info()
    n_workers = info.num_cores * info.num_subcores
    g_i32 = _sc_gather_build(n_rows, D, n_workers, info.num_cores)(
        packed, idx_flat)

    g = (lax.bitcast_convert_type(g_i32, jnp.float16)
         .reshape(B, S, 2 * D).astype(jnp.float32))

    out_shape = jax.ShapeDtypeStruct((B, H, S, D), jnp.float32)
    ocos, osin = pl.pallas_call(
        _bcast_body,
        grid=(B, H),
        in_specs=[
            pl.BlockSpec((B, S), lambda b, h: (0, 0)),
            pl.BlockSpec((1, half_words), lambda b, h: (0, 0)),
            pl.BlockSpec((1, S, 2 * D), lambda b, h: (b, 0, 0)),
        ],
        out_specs=[
            pl.BlockSpec((1, 1, S, D), lambda b, h: (b, h, 0, 0)),
            pl.BlockSpec((1, 1, S, D), lambda b, h: (b, h, 0, 0)),
        ],
        out_shape=[out_shape, out_shape],
        compiler_params=pltpu.CompilerParams(
            dimension_semantics=("arbitrary", "arbitrary")),
    )(pos, inv_freq.reshape(1, half_words), g)
    return ocos.astype(x.dtype), osin.astype(x.dtype)


# speedup vs baseline: 1.5024x; 1.5024x over previous
import functools

import jax
import jax.numpy as jnp
from jax import lax
from jax.experimental import pallas as pl
from jax.experimental.pallas import tpu as pltpu
from jax.experimental.pallas import tpu_sc as plsc

_TABLE_SIZE = 2048


def _sc_gather_build(n_rows, row_words, n_workers, nc):
    rows_per_w = n_rows // n_workers
    mesh = plsc.VectorSubcoreMesh(core_axis_name="c", subcore_axis_name="s")

    @functools.partial(
        pl.kernel,
        out_type=(
            jax.ShapeDtypeStruct((n_rows, row_words), jnp.float32),
            jax.ShapeDtypeStruct((n_rows, row_words), jnp.float32),
        ),
        mesh=mesh,
        scratch_types=[
            pltpu.VMEM((rows_per_w,), jnp.int32),
            pltpu.VMEM((rows_per_w, row_words), jnp.float32),
            pltpu.VMEM((rows_per_w, row_words), jnp.float32),
            pltpu.SemaphoreType.DMA,
        ],
    )
    def sc_gather(cos_hbm, sin_hbm, idx_hbm, out_cos, out_sin,
                  idx_v, rows_c, rows_s, sem):
        wid = lax.axis_index("s") * nc + lax.axis_index("c")
        base = wid * rows_per_w
        pltpu.sync_copy(idx_hbm.at[pl.ds(base, rows_per_w)], idx_v)
        cc = pltpu.make_async_copy(cos_hbm.at[idx_v], rows_c, sem)
        cs = pltpu.make_async_copy(sin_hbm.at[idx_v], rows_s, sem)
        cc.start()
        cs.start()
        cc.wait()
        cs.wait()
        pltpu.sync_copy(rows_c, out_cos.at[pl.ds(base, rows_per_w)])
        pltpu.sync_copy(rows_s, out_sin.at[pl.ds(base, rows_per_w)])

    return sc_gather


def _fanout_body(ids_ref, inv_ref, gcos_ref, gsin_ref, ocos_ref, osin_ref,
                 scos, ssin, sem):
    ids = ids_ref[...]
    in_table = jnp.logical_and(jnp.max(ids) < _TABLE_SIZE, jnp.min(ids) >= 0)
    scale = jnp.where(in_table, jnp.float32(1.0),
                      jnp.float32(1.0) + jnp.sum(inv_ref[...]))
    scos[...] = gcos_ref[...] * scale
    ssin[...] = gsin_ref[...] * scale
    B, H = ocos_ref.shape[0], ocos_ref.shape[1]
    copies = []
    for b in range(B):
        for h in range(H):
            copies.append(pltpu.make_async_copy(
                scos.at[b], ocos_ref.at[b, h], sem))
            copies.append(pltpu.make_async_copy(
                ssin.at[b], osin_ref.at[b, h], sem))
    for c in copies:
        c.start()
    for c in copies:
        c.wait()


def kernel(x, lookup_cos, lookup_sin, inv_freq, position_ids):
    B, H, S, D = x.shape
    T = lookup_cos.shape[0]
    pos = position_ids.astype(jnp.int32)
    n_rows = B * S

    cos_f32 = lookup_cos.astype(jnp.float32)
    sin_f32 = lookup_sin.astype(jnp.float32)
    idx_flat = jnp.clip(pos.reshape(n_rows), 0, T - 1)

    info = plsc.get_sparse_core_info()
    n_workers = info.num_cores * info.num_subcores
    g_cos, g_sin = _sc_gather_build(n_rows, D, n_workers, info.num_cores)(
        cos_f32, sin_f32, idx_flat)
    g_cos = g_cos.reshape(B, S, D)
    g_sin = g_sin.reshape(B, S, D)

    out_shape = jax.ShapeDtypeStruct((B, H, S, D), jnp.float32)
    ocos, osin = pl.pallas_call(
        _fanout_body,
        in_specs=[
            pl.BlockSpec((B, S), lambda: (0, 0)),
            pl.BlockSpec((1, D // 2), lambda: (0, 0)),
            pl.BlockSpec((B, S, D), lambda: (0, 0, 0)),
            pl.BlockSpec((B, S, D), lambda: (0, 0, 0)),
        ],
        out_specs=[
            pl.BlockSpec(memory_space=pl.ANY),
            pl.BlockSpec(memory_space=pl.ANY),
        ],
        out_shape=[out_shape, out_shape],
        scratch_shapes=[
            pltpu.VMEM((B, S, D), jnp.float32),
            pltpu.VMEM((B, S, D), jnp.float32),
            pltpu.SemaphoreType.DMA,
        ],
    )(pos, inv_freq.reshape(1, D // 2), g_cos, g_sin)
    return ocos.astype(x.dtype), osin.astype(x.dtype)
